# baseline (speedup 1.0000x reference)
import functools

import jax
import jax.numpy as jnp
from jax import lax
from jax.experimental import pallas as pl
from jax.experimental.pallas import tpu as pltpu
from jax.experimental.pallas import tpu_sc as plsc


def kernel(idx, table):
    B, = idx.shape
    V, D = table.shape

    info = plsc.get_sparse_core_info()
    NC, NS, L = info.num_cores, info.num_subcores, info.num_lanes
    NW = NC * NS
    b_per_w = B // NW
    U = 8
    Vw = V // 8 * 8

    mesh = plsc.VectorSubcoreMesh(core_axis_name="c", subcore_axis_name="s")

    @functools.partial(
        pl.kernel,
        mesh=mesh,
        out_type=jax.ShapeDtypeStruct((B, D), jnp.float32),
        scratch_types=[
            pltpu.SMEM((b_per_w,), jnp.int32),
            pltpu.VMEM((V, D), jnp.float32),
            pltpu.VMEM_SHARED((Vw, D), jnp.float32),
            pltpu.VMEM_SHARED((B,), jnp.int32),
            pltpu.SemaphoreType.DMA,
        ],
    )
    def gather_kernel(idx_hbm, table_hbm, out_hbm, idx_m, table_v,
                      table_s, idx_s, sem):
        sid = lax.axis_index("s")
        wid = sid * NC + lax.axis_index("c")
        base = wid * b_per_w

        @pl.when(sid == 0)
        def _stage_shared():
            pltpu.sync_copy(table_hbm.at[pl.ds(0, Vw)], table_s)
            pltpu.sync_copy(idx_hbm, idx_s)

        pltpu.sync_copy(table_hbm.at[pl.ds(Vw, V - Vw)],
                        table_v.at[pl.ds(Vw, V - Vw)])
        plsc.subcore_barrier()
        pltpu.sync_copy(idx_s.at[pl.ds(base, b_per_w)], idx_m)
        pltpu.sync_copy(table_s, table_v.at[pl.ds(0, Vw)])

        def body(g, carry):
            i0 = g * U
            for u in range(U):
                row = idx_m[i0 + u]
                pltpu.async_copy(
                    table_v.at[row], out_hbm.at[base + i0 + u], sem)
            return carry

        lax.fori_loop(0, b_per_w // U, body, 0)

        for _ in range(b_per_w // 128):
            pltpu.make_async_copy(
                table_v.at[pl.ds(0, 128)],
                out_hbm.at[pl.ds(base, 128)], sem).wait()

    return gather_kernel(idx.astype(jnp.int32), table)


# speedup vs baseline: 1.0274x; 1.0274x over previous
import functools

import jax
import jax.numpy as jnp
from jax import lax
from jax.experimental import pallas as pl
from jax.experimental.pallas import tpu as pltpu
from jax.experimental.pallas import tpu_sc as plsc


def kernel(idx, table):
    B, = idx.shape
    V, D = table.shape

    info = plsc.get_sparse_core_info()
    NC, NS, L = info.num_cores, info.num_subcores, info.num_lanes
    NW = NC * NS
    b_per_w = B // NW
    U = 8
    Vp = (V + 7) // 8 * 8

    mesh = plsc.VectorSubcoreMesh(core_axis_name="c", subcore_axis_name="s")

    @functools.partial(
        pl.kernel,
        mesh=mesh,
        out_type=jax.ShapeDtypeStruct((B, D), jnp.float32),
        scratch_types=[
            pltpu.SMEM((b_per_w,), jnp.int32),
            pltpu.VMEM((Vp, D), jnp.float32),
            pltpu.VMEM_SHARED((Vp, D), jnp.float32),
            pltpu.VMEM_SHARED((B,), jnp.int32),
            pltpu.SemaphoreType.DMA,
        ],
    )
    def gather_kernel(idx_hbm, table_hbm, out_hbm, idx_m, table_v,
                      table_s, idx_s, sem):
        sid = lax.axis_index("s")
        wid = sid * NC + lax.axis_index("c")
        base = wid * b_per_w

        @pl.when(sid == 0)
        def _stage_shared():
            pltpu.sync_copy(table_hbm, table_s)
            pltpu.sync_copy(idx_hbm, idx_s)

        plsc.subcore_barrier()
        pltpu.sync_copy(table_s, table_v)
        pltpu.sync_copy(idx_s.at[pl.ds(base, b_per_w)], idx_m)

        def body(g, carry):
            i0 = g * U
            for u in range(U):
                row = idx_m[i0 + u]
                pltpu.async_copy(
                    table_v.at[row], out_hbm.at[base + i0 + u], sem)
            return carry

        lax.fori_loop(0, b_per_w // U, body, 0)

        for _ in range(b_per_w // 128):
            pltpu.make_async_copy(
                table_v.at[pl.ds(0, 128)],
                out_hbm.at[pl.ds(base, 128)], sem).wait()

    table_p = jnp.pad(table, ((0, Vp - V), (0, 0)))
    return gather_kernel(idx.astype(jnp.int32), table_p)


# speedup vs baseline: 1.0351x; 1.0075x over previous
import functools

import jax
import jax.numpy as jnp
from jax import lax
from jax.experimental import pallas as pl
from jax.experimental.pallas import tpu as pltpu
from jax.experimental.pallas import tpu_sc as plsc


def kernel(idx, table):
    B, = idx.shape
    V, D = table.shape

    info = plsc.get_sparse_core_info()
    NC, NS, L = info.num_cores, info.num_subcores, info.num_lanes
    NW = NC * NS
    b_per_w = B // NW
    U = 4
    Vp = (V + 7) // 8 * 8

    mesh = plsc.VectorSubcoreMesh(core_axis_name="c", subcore_axis_name="s")

    @functools.partial(
        pl.kernel,
        mesh=mesh,
        out_type=jax.ShapeDtypeStruct((B, D), jnp.float32),
        scratch_types=[
            pltpu.SMEM((b_per_w,), jnp.int32),
            pltpu.VMEM((Vp, D), jnp.float32),
            pltpu.VMEM_SHARED((Vp, D), jnp.float32),
            pltpu.VMEM_SHARED((B,), jnp.int32),
            pltpu.SemaphoreType.DMA,
        ],
    )
    def gather_kernel(idx_hbm, table_hbm, out_hbm, idx_m, table_v,
                      table_s, idx_s, sem):
        sid = lax.axis_index("s")
        wid = sid * NC + lax.axis_index("c")
        base = wid * b_per_w

        @pl.when(sid == 0)
        def _stage_shared():
            pltpu.sync_copy(table_hbm, table_s)
            pltpu.sync_copy(idx_hbm, idx_s)

        plsc.subcore_barrier()
        pltpu.sync_copy(table_s, table_v)
        pltpu.sync_copy(idx_s.at[pl.ds(base, b_per_w)], idx_m)

        def body(g, carry):
            i0 = g * U
            for u in range(U):
                row = idx_m[i0 + u]
                pltpu.async_copy(
                    table_v.at[row], out_hbm.at[base + i0 + u], sem)
            return carry

        lax.fori_loop(0, b_per_w // U, body, 0)

        for _ in range(b_per_w // 128):
            pltpu.make_async_copy(
                table_v.at[pl.ds(0, 128)],
                out_hbm.at[pl.ds(base, 128)], sem).wait()

    table_p = jnp.pad(table, ((0, Vp - V), (0, 0)))
    return gather_kernel(idx.astype(jnp.int32), table_p)
